# baseline (speedup 1.0000x reference)
import functools

import jax
import jax.numpy as jnp
from jax import lax
from jax.experimental import pallas as pl
from jax.experimental.pallas import tpu as pltpu
from jax.experimental.pallas import tpu_sc as plsc

_CHUNK = 512
_NBUF = 2


def _eye(n):
    r = lax.broadcasted_iota(jnp.int32, (n, n), 0)
    c = lax.broadcasted_iota(jnp.int32, (n, n), 1)
    return jnp.where(r == c, 1.0, 0.0).astype(jnp.float32)


def _transpose_table(tt):
    C, V = tt.shape
    BT = 2048

    def body(in_ref, out_ref):
        a = in_ref[...]
        out_ref[...] = lax.dot_general(
            a, _eye(C), (((0,), (0,)), ((), ())),
            preferred_element_type=jnp.float32)

    return pl.pallas_call(
        body,
        grid=(pl.cdiv(V, BT),),
        in_specs=[pl.BlockSpec((C, BT), lambda b: (0, b))],
        out_specs=pl.BlockSpec((BT, C), lambda b: (b, 0)),
        out_shape=jax.ShapeDtypeStruct((V, C), jnp.float32),
    )(tt)


def _transpose_out(g, J, I):
    C = g.shape[1]
    BI = 512
    assert I % BI == 0

    def body(in_ref, out_ref):
        a = in_ref[0]
        out_ref[0] = lax.dot_general(
            _eye(C), a, (((0,), (1,)), ((), ())),
            preferred_element_type=jnp.float32)

    return pl.pallas_call(
        body,
        grid=(J, I // BI),
        in_specs=[pl.BlockSpec((1, BI, C), lambda j, b: (j, b, 0))],
        out_specs=pl.BlockSpec((1, C, BI), lambda j, b: (j, 0, b)),
        out_shape=jax.ShapeDtypeStruct((J, C, I), jnp.float32),
    )(g.reshape(J, I, C))


def _sc_gather(table, idx_flat):
    V, D = table.shape
    B = idx_flat.shape[0]
    info = plsc.get_sparse_core_info()
    NC, NS = info.num_cores, info.num_subcores
    NW = NC * NS
    b_per_w = B // NW
    n_chunks = b_per_w // _CHUNK
    n_groups = n_chunks // _NBUF
    assert b_per_w * NW == B and n_chunks * _CHUNK == b_per_w
    assert n_groups * _NBUF == n_chunks and n_groups >= 2

    mesh = plsc.VectorSubcoreMesh(core_axis_name="c", subcore_axis_name="s")

    @functools.partial(
        pl.kernel,
        mesh=mesh,
        out_type=jax.ShapeDtypeStruct((B, D), jnp.float32),
        scratch_types=(
            [pltpu.VMEM((b_per_w,), jnp.int32),
             pltpu.VMEM((_NBUF, _CHUNK, D), jnp.float32)]
            + [pltpu.SemaphoreType.DMA] * (2 * _NBUF)
        ),
        compiler_params=pltpu.CompilerParams(use_tc_tiling_on_sc=False),
    )
    def k(table_hbm, idx_hbm, out_hbm, idx_v, rows_v, *sems):
        gsem, wsem = sems[:_NBUF], sems[_NBUF:]
        wid = lax.axis_index("s") * NC + lax.axis_index("c")
        base = wid * b_per_w
        pltpu.sync_copy(idx_hbm.at[pl.ds(base, b_per_w)], idx_v)

        def gather_desc(j, b):
            return pltpu.make_async_copy(
                table_hbm.at[idx_v.at[pl.ds(j * _CHUNK, _CHUNK)]],
                rows_v.at[b], gsem[b])

        def write_desc(j, b):
            return pltpu.make_async_copy(
                rows_v.at[b], out_hbm.at[pl.ds(base + j * _CHUNK, _CHUNK)],
                wsem[b])

        for b in range(_NBUF):
            gather_desc(b, b).start()

        def body(g, carry):
            j0 = g * _NBUF
            for b in range(_NBUF):
                gather_desc(j0 + b, b).wait()
                write_desc(j0 + b, b).start()
            for b in range(_NBUF):
                write_desc(j0 + b, b).wait()
                gather_desc(j0 + _NBUF + b, b).start()
            return carry

        lax.fori_loop(0, n_groups - 1, body, 0, unroll=False)

        jf = (n_groups - 1) * _NBUF
        for b in range(_NBUF):
            gather_desc(jf + b, b).wait()
            write_desc(jf + b, b).start()
        for b in range(_NBUF):
            write_desc(jf + b, b).wait()

    return k(table, idx_flat)


@jax.jit
def _run(input_, indices):
    V, D = input_.shape
    I, J = indices.shape
    table = _transpose_table(input_.T)
    idx_flat = indices.T.reshape(I * J)
    g = _sc_gather(table, idx_flat)
    out_t = _transpose_out(g, J, I)
    return out_t.transpose(2, 0, 1)


def kernel(input_, indices):
    return _run(input_, indices)


# speedup vs baseline: 1.6532x; 1.6532x over previous
import functools

import jax
import jax.numpy as jnp
from jax import lax
from jax.experimental import pallas as pl
from jax.experimental.pallas import tpu as pltpu
from jax.experimental.pallas import tpu_sc as plsc

_C = 64


@jax.jit
def _gather_flat(table2, idx):
    R, TW = table2.shape
    D = TW // 2
    (B,) = idx.shape
    info = plsc.get_sparse_core_info()
    NC = info.num_cores
    NW = NC * info.num_subcores
    b_per_w = B // NW
    assert B == NW * b_per_w and b_per_w % (4 * _C) == 0
    n = b_per_w // _C

    mesh = plsc.VectorSubcoreMesh(core_axis_name="c", subcore_axis_name="s")

    @functools.partial(
        pl.kernel,
        mesh=mesh,
        out_type=jax.ShapeDtypeStruct((B, D), jnp.float32),
        scratch_types=[
            [pltpu.VMEM((_C,), jnp.int32)] * 4,
            [pltpu.VMEM((_C,), jnp.int32)] * 2,
            [pltpu.VMEM((_C,), jnp.int32)] * 2,
            [pltpu.VMEM((_C, TW), jnp.float32)] * 2,
            [pltpu.VMEM((_C, D), jnp.float32)] * 2,
            [pltpu.SemaphoreType.DMA] * 4,
            [pltpu.SemaphoreType.DMA] * 2,
            [pltpu.SemaphoreType.DMA] * 2,
        ],
        compiler_params=pltpu.CompilerParams(
            use_tc_tiling_on_sc=True, needs_layout_passes=False),
    )
    def k(tbl, idxs, out, idx_v, i2, par, rows, dsel, is_, gs, ws):
        wid = lax.axis_index("s") * NC + lax.axis_index("c")
        base = wid * b_per_w
        iota16 = lax.iota(jnp.int32, 16)

        def idesc(j, s):
            return pltpu.make_async_copy(
                idxs.at[pl.ds(base + j * _C, _C)], idx_v[s], is_[s])

        def gdesc(p):
            return pltpu.make_async_copy(tbl.at[i2[p]], rows[p], gs[p])

        def wdesc(j, p):
            return pltpu.make_async_copy(
                dsel[p], out.at[pl.ds(base + j * _C, _C)], ws[p])

        def prep_and_fire(j, s, p):
            idesc(j, s).wait()
            for kb in range(_C // 16):
                v = idx_v[s][pl.ds(kb * 16, 16)]
                i2[p][pl.ds(kb * 16, 16)] = jnp.right_shift(v, 1)
                par[p][pl.ds(kb * 16, 16)] = jnp.bitwise_and(v, 1) * D
            gdesc(p).start()

        def select(p):
            for rb in range(_C // 16):
                rid = iota16 + (rb * 16)
                p64 = par[p][pl.ds(rb * 16, 16)]
                for cb in range(D // 16):
                    c0 = cb * 16
                    for kk in range(16):
                        cc = jnp.bitwise_and(iota16 + kk, 15) + c0
                        vals = plsc.load_gather(rows[p], [rid, cc + p64])
                        plsc.store_scatter(dsel[p], [rid, cc], vals)

        for j in range(4):
            idesc(j, j).start()
        prep_and_fire(0, 0, 0)
        prep_and_fire(1, 1, 1)

        def body(g, carry):
            ge = lax.rem(g, 2)
            for p in (0, 1):
                j = 2 * g + p
                gdesc(p).wait()

                @pl.when(j >= 2)
                def _():
                    wdesc(j - 2, p).wait()

                select(p)
                wdesc(j, p).start()

                @pl.when(j + 4 < n)
                def _():
                    pl.when(ge == 0)(lambda: idesc(j + 4, p).start())
                    pl.when(ge == 1)(lambda: idesc(j + 4, p + 2).start())

                @pl.when(j + 2 < n)
                def _():
                    pl.when(ge == 0)(lambda: prep_and_fire(j + 2, p + 2, p))
                    pl.when(ge == 1)(lambda: prep_and_fire(j + 2, p, p))

            return carry

        lax.fori_loop(0, n // 2, body, 0, unroll=False)
        wdesc(n - 1, 1).wait()
        wdesc(n - 2, 0).wait()

    return k(table2, idx)


def kernel(input_, indices):
    V, D = input_.shape
    I, J = indices.shape
    table2 = input_.reshape(V // 2, 2 * D)
    flat = _gather_flat(table2, indices.reshape(I * J))
    return flat.reshape(I, J, D)
